# baseline (speedup 1.0000x reference)
import jax
import jax.numpy as jnp
from jax import lax
from jax.experimental import pallas as pl
from jax.experimental.pallas import tpu as pltpu
from jax.experimental.pallas import tpu_sc as plsc

N_NODES = 100000
N_EDGES = 1600000
NPAD = 102400
NC, NS = 2, 16
NW = NC * NS
PER_W = N_EDGES // NW
CHUNK = 2000
NCHUNK = PER_W // CHUNK
SLICE = NPAD // NS

_f32 = jnp.float32


def _mesh():
    return plsc.VectorSubcoreMesh(
        core_axis_name="c", subcore_axis_name="s", num_cores=NC, num_subcores=NS
    )


DEG_PER_TILE = N_EDGES // NS
DEG_NCHUNK = DEG_PER_TILE // CHUNK


def _nrsqrt16(x):
    i = lax.bitcast_convert_type(x, jnp.int32)
    i = 0x5F3759DF - (i >> 1)
    y = lax.bitcast_convert_type(i, _f32)
    for _ in range(3):
        y = y * (1.5 - 0.5 * x * y * y)
    return y


def _l1_body(src_hbm, dst_hbm, x0_hbm, x1_hbm, zeros_hbm, ones_hbm,
             out_hbm, dinv_hbm,
             dacc, tab0, tab1, acc0, acc1, srcv0, srcv1, dstv0, dstv1,
             v00, v01, v10, v11, onesv, vdeg, vx0, vx1,
             sem_i, sem_g0, sem_g1, sem_s0, sem_s1):
    c = lax.axis_index("c")
    s = lax.axis_index("s")
    w = s * NC + c
    sl = pl.ds(s * SLICE, SLICE)
    dstv = (dstv0, dstv1)
    sem_s = (sem_s0, sem_s1)

    pltpu.sync_copy(zeros_hbm, dacc.at[sl])
    pltpu.sync_copy(ones_hbm, onesv)
    plsc.subcore_barrier()
    d_d = [None] * DEG_NCHUNK
    for k in range(DEG_NCHUNK):
        b = k % 2
        if k >= 2:
            d_d[k - 2].wait()
        base = s * DEG_PER_TILE + k * CHUNK
        i_d = pltpu.async_copy(dst_hbm.at[pl.ds(base, CHUNK)], dstv[b], sem_i)
        i_d.wait()
        d_d[k] = pltpu.async_copy(onesv, dacc.at[dstv[b]], sem_s[b], add=True)
    d_d[DEG_NCHUNK - 2].wait()
    d_d[DEG_NCHUNK - 1].wait()
    plsc.subcore_barrier()

    pltpu.sync_copy(dacc.at[sl], vdeg)
    pltpu.sync_copy(x0_hbm.at[sl], vx0)
    pltpu.sync_copy(x1_hbm.at[sl], vx1)

    def pbody(i, carry):
        ds16 = pl.ds(i * 16, 16)
        dv = _nrsqrt16(vdeg[ds16] + 1.0)
        vdeg[ds16] = dv
        vx0[ds16] = dv * vx0[ds16]
        vx1[ds16] = dv * vx1[ds16]
        return carry

    lax.fori_loop(0, SLICE // 16, pbody, 0)
    pltpu.sync_copy(vx0, tab0.at[sl])
    pltpu.sync_copy(vx1, tab1.at[sl])

    @pl.when(c == 0)
    def _():
        pltpu.sync_copy(vdeg, dinv_hbm.at[sl])

    pltpu.sync_copy(zeros_hbm, acc0.at[sl])
    pltpu.sync_copy(zeros_hbm, acc1.at[sl])
    plsc.subcore_barrier()

    srcv = (srcv0, srcv1)
    dstv = (dstv0, dstv1)
    v0 = (v00, v01)
    v1 = (v10, v11)
    sem_g = (sem_g0, sem_g1)
    sem_s = (sem_s0, sem_s1)

    g_d = [None] * NCHUNK
    s_d = [None] * NCHUNK
    for k in range(NCHUNK):
        b = k % 2
        if k >= 2:
            for d in s_d[k - 2]:
                d.wait()
        base = w * PER_W + k * CHUNK
        ia = pltpu.async_copy(src_hbm.at[pl.ds(base, CHUNK)], srcv[b], sem_i)
        ib = pltpu.async_copy(dst_hbm.at[pl.ds(base, CHUNK)], dstv[b], sem_i)
        ia.wait()
        ib.wait()
        g_d[k] = (pltpu.async_copy(tab0.at[srcv[b]], v0[b], sem_g[b]),
                  pltpu.async_copy(tab1.at[srcv[b]], v1[b], sem_g[b]))
        if k >= 1:
            p = (k - 1) % 2
            for d in g_d[k - 1]:
                d.wait()
            s_d[k - 1] = (
                pltpu.async_copy(v0[p], acc0.at[dstv[p]], sem_s[p], add=True),
                pltpu.async_copy(v1[p], acc1.at[dstv[p]], sem_s[p], add=True))
    b = (NCHUNK - 1) % 2
    for d in g_d[NCHUNK - 1]:
        d.wait()
    s_d[NCHUNK - 1] = (
        pltpu.async_copy(v0[b], acc0.at[dstv[b]], sem_s[b], add=True),
        pltpu.async_copy(v1[b], acc1.at[dstv[b]], sem_s[b], add=True))
    for d in s_d[NCHUNK - 2]:
        d.wait()
    for d in s_d[NCHUNK - 1]:
        d.wait()
    plsc.subcore_barrier()
    pltpu.sync_copy(acc0.at[sl], out_hbm.at[c, 0, sl])
    pltpu.sync_copy(acc1.at[sl], out_hbm.at[c, 1, sl])


def _l2_body(src_hbm, dst_hbm, t0_hbm, zeros_hbm, out_hbm,
             tab, acc0, srcv0, srcv1, dstv0, dstv1, v00, v01,
             sem_i, sem_g0, sem_g1, sem_s0, sem_s1):
    c = lax.axis_index("c")
    s = lax.axis_index("s")
    w = s * NC + c
    sl = pl.ds(s * SLICE, SLICE)
    pltpu.sync_copy(t0_hbm.at[sl], tab.at[sl])
    pltpu.sync_copy(zeros_hbm, acc0.at[sl])
    plsc.subcore_barrier()

    srcv = (srcv0, srcv1)
    dstv = (dstv0, dstv1)
    v0 = (v00, v01)
    sem_g = (sem_g0, sem_g1)
    sem_s = (sem_s0, sem_s1)

    g_d = [None] * NCHUNK
    s_d = [None] * NCHUNK
    for k in range(NCHUNK):
        b = k % 2
        if k >= 2:
            s_d[k - 2].wait()
        base = w * PER_W + k * CHUNK
        ia = pltpu.async_copy(src_hbm.at[pl.ds(base, CHUNK)], srcv[b], sem_i)
        ib = pltpu.async_copy(dst_hbm.at[pl.ds(base, CHUNK)], dstv[b], sem_i)
        ia.wait()
        ib.wait()
        g_d[k] = pltpu.async_copy(tab.at[srcv[b]], v0[b], sem_g[b])
        if k >= 1:
            p = (k - 1) % 2
            g_d[k - 1].wait()
            s_d[k - 1] = pltpu.async_copy(v0[p], acc0.at[dstv[p]], sem_s[p], add=True)
    b = (NCHUNK - 1) % 2
    g_d[NCHUNK - 1].wait()
    s_d[NCHUNK - 1] = pltpu.async_copy(v0[b], acc0.at[dstv[b]], sem_s[b], add=True)
    s_d[NCHUNK - 2].wait()
    s_d[NCHUNK - 1].wait()
    plsc.subcore_barrier()
    pltpu.sync_copy(acc0.at[sl], out_hbm.at[c, sl])


_l1_call = pl.kernel(
    _l1_body,
    out_type=(jax.ShapeDtypeStruct((NC, 2, NPAD), _f32),
              jax.ShapeDtypeStruct((NPAD,), _f32)),
    mesh=_mesh(),
    scratch_types=[
        pltpu.VMEM_SHARED((NPAD,), _f32),
        pltpu.VMEM_SHARED((NPAD,), _f32),
        pltpu.VMEM_SHARED((NPAD,), _f32),
        pltpu.VMEM_SHARED((NPAD,), _f32),
        pltpu.VMEM_SHARED((NPAD,), _f32),
        pltpu.VMEM((CHUNK,), jnp.int32),
        pltpu.VMEM((CHUNK,), jnp.int32),
        pltpu.VMEM((CHUNK,), jnp.int32),
        pltpu.VMEM((CHUNK,), jnp.int32),
        pltpu.VMEM((CHUNK,), _f32),
        pltpu.VMEM((CHUNK,), _f32),
        pltpu.VMEM((CHUNK,), _f32),
        pltpu.VMEM((CHUNK,), _f32),
        pltpu.VMEM((CHUNK,), _f32),
        pltpu.VMEM((SLICE,), _f32),
        pltpu.VMEM((SLICE,), _f32),
        pltpu.VMEM((SLICE,), _f32),
        pltpu.SemaphoreType.DMA,
        pltpu.SemaphoreType.DMA,
        pltpu.SemaphoreType.DMA,
        pltpu.SemaphoreType.DMA,
        pltpu.SemaphoreType.DMA,
    ],
)

_l2_call = pl.kernel(
    _l2_body,
    out_type=jax.ShapeDtypeStruct((NC, NPAD), _f32),
    mesh=_mesh(),
    scratch_types=[
        pltpu.VMEM_SHARED((NPAD,), _f32),
        pltpu.VMEM_SHARED((NPAD,), _f32),
        pltpu.VMEM((CHUNK,), jnp.int32),
        pltpu.VMEM((CHUNK,), jnp.int32),
        pltpu.VMEM((CHUNK,), jnp.int32),
        pltpu.VMEM((CHUNK,), jnp.int32),
        pltpu.VMEM((CHUNK,), _f32),
        pltpu.VMEM((CHUNK,), _f32),
        pltpu.SemaphoreType.DMA,
        pltpu.SemaphoreType.DMA,
        pltpu.SemaphoreType.DMA,
        pltpu.SemaphoreType.DMA,
        pltpu.SemaphoreType.DMA,
    ],
)


def _tc2_body(a1p, x0, x1, dinv, w1t, b1, w2t, z_o, y2_o):
    dv = dinv[...]
    d2 = dv * dv
    ap = a1p[0] + a1p[1]
    xx = jnp.concatenate([x0[...], x1[...]], axis=0)
    ax = dv * ap + d2 * xx
    h = jnp.dot(w1t[...], ax, preferred_element_type=_f32) + b1[...]
    h = jnp.maximum(h, 0.0)
    z = jnp.dot(w2t[...], h, preferred_element_type=_f32)
    z_o[...] = z
    y2_o[...] = dv * z


def _tc3_body(a2p, z, dinv, b2, out_o):
    dv = dinv[...]
    out_o[...] = dv * (a2p[0:1, :] + a2p[1:2, :] + dv * z[...]) + b2[...]


_tc2_call = pl.pallas_call(
    _tc2_body,
    out_shape=(
        jax.ShapeDtypeStruct((1, NPAD), _f32),
        jax.ShapeDtypeStruct((1, NPAD), _f32),
    ),
)

_tc3_call = pl.pallas_call(
    _tc3_body,
    out_shape=jax.ShapeDtypeStruct((1, NPAD), _f32),
)


def kernel(x, edge_index, W1, b1, W2, b2):
    src = edge_index[0].astype(jnp.int32)
    dst = edge_index[1].astype(jnp.int32)
    pad = NPAD - N_NODES
    x0 = jnp.pad(x[:, 0], (0, pad)).reshape(1, NPAD)
    x1 = jnp.pad(x[:, 1], (0, pad)).reshape(1, NPAD)
    zeros_h = jnp.zeros((SLICE,), _f32)
    ones_h = jnp.ones((CHUNK,), _f32)
    w1t = W1.T
    w2t = W2.T
    b1c = b1.reshape(64, 1)
    b2c = b2.reshape(1, 1)

    a1p, dinv = _l1_call(src, dst, x0.reshape(NPAD), x1.reshape(NPAD),
                         zeros_h, ones_h)
    dinv2 = dinv.reshape(1, NPAD)
    z, y2 = _tc2_call(a1p, x0, x1, dinv2, w1t, b1c, w2t)
    a2p = _l2_call(src, dst, y2.reshape(NPAD), zeros_h)
    out = _tc3_call(a2p, z, dinv2, b2c)
    return out.reshape(NPAD)[:N_NODES]


# speedup vs baseline: 1.1664x; 1.1664x over previous
import jax
import jax.numpy as jnp
from jax import lax
from jax.experimental import pallas as pl
from jax.experimental.pallas import tpu as pltpu
from jax.experimental.pallas import tpu_sc as plsc

N_NODES = 100000
N_EDGES = 1600000
NPAD = 102400
NC, NS = 2, 16
NW = NC * NS
PER_W = N_EDGES // NW
CHUNK = 2000
NCHUNK = PER_W // CHUNK
SLICE = NPAD // NS

_f32 = jnp.float32


def _mesh():
    return plsc.VectorSubcoreMesh(
        core_axis_name="c", subcore_axis_name="s", num_cores=NC, num_subcores=NS
    )


DEG_PER_TILE = N_EDGES // NS
DEG_NCHUNK = DEG_PER_TILE // CHUNK


def _nrsqrt16(x):
    i = lax.bitcast_convert_type(x, jnp.int32)
    i = 0x5F3759DF - (i >> 1)
    y = lax.bitcast_convert_type(i, _f32)
    for _ in range(3):
        y = y * (1.5 - 0.5 * x * y * y)
    return y


def _l1_body(src_hbm, dst_hbm, x0_hbm, x1_hbm, zeros_hbm, ones_hbm,
             out_hbm, dinv_hbm,
             dacc, tab0, tab1, acc0, acc1,
             srcv0, srcv1, srcv2, dstv0, dstv1, dstv2,
             v00, v01, v10, v11, onesv, vdeg, vx0, vx1,
             sem_i, sem_g0, sem_g1, sem_s0, sem_s1):
    c = lax.axis_index("c")
    s = lax.axis_index("s")
    w = s * NC + c
    sl = pl.ds(s * SLICE, SLICE)
    dbuf = (dstv0, dstv1, dstv2)
    sem_s = (sem_s0, sem_s1)

    pltpu.sync_copy(zeros_hbm, dacc.at[sl])
    pltpu.sync_copy(ones_hbm, onesv)
    plsc.subcore_barrier()
    i_d = [None] * DEG_NCHUNK
    d_d = [None] * DEG_NCHUNK
    i_d[0] = pltpu.async_copy(
        dst_hbm.at[pl.ds(s * DEG_PER_TILE, CHUNK)], dbuf[0], sem_i)
    for k in range(DEG_NCHUNK):
        i_d[k].wait()
        if k >= 2:
            d_d[k - 2].wait()
        if k + 1 < DEG_NCHUNK:
            base = s * DEG_PER_TILE + (k + 1) * CHUNK
            i_d[k + 1] = pltpu.async_copy(
                dst_hbm.at[pl.ds(base, CHUNK)], dbuf[(k + 1) % 3], sem_i)
        d_d[k] = pltpu.async_copy(onesv, dacc.at[dbuf[k % 3]], sem_s[k % 2],
                                  add=True)
    d_d[DEG_NCHUNK - 2].wait()
    d_d[DEG_NCHUNK - 1].wait()
    plsc.subcore_barrier()

    pltpu.sync_copy(dacc.at[sl], vdeg)
    pltpu.sync_copy(x0_hbm.at[sl], vx0)
    pltpu.sync_copy(x1_hbm.at[sl], vx1)

    def pbody(i, carry):
        ds16 = pl.ds(i * 16, 16)
        dv = _nrsqrt16(vdeg[ds16] + 1.0)
        vdeg[ds16] = dv
        vx0[ds16] = dv * vx0[ds16]
        vx1[ds16] = dv * vx1[ds16]
        return carry

    lax.fori_loop(0, SLICE // 16, pbody, 0)
    pltpu.sync_copy(vx0, tab0.at[sl])
    pltpu.sync_copy(vx1, tab1.at[sl])

    @pl.when(c == 0)
    def _():
        pltpu.sync_copy(vdeg, dinv_hbm.at[sl])

    pltpu.sync_copy(zeros_hbm, acc0.at[sl])
    pltpu.sync_copy(zeros_hbm, acc1.at[sl])
    plsc.subcore_barrier()

    srcv = (srcv0, srcv1, srcv2)
    dstv = (dstv0, dstv1, dstv2)
    v0 = (v00, v01)
    v1 = (v10, v11)
    sem_g = (sem_g0, sem_g1)
    sem_s = (sem_s0, sem_s1)

    i_as = [None] * NCHUNK
    i_bs = [None] * NCHUNK
    g_d = [None] * NCHUNK
    s_d = [None] * NCHUNK
    base0 = w * PER_W
    i_as[0] = pltpu.async_copy(src_hbm.at[pl.ds(base0, CHUNK)], srcv[0], sem_i)
    i_bs[0] = pltpu.async_copy(dst_hbm.at[pl.ds(base0, CHUNK)], dstv[0], sem_i)
    for k in range(NCHUNK):
        vb = k % 2
        if k >= 2:
            for d in s_d[k - 2]:
                d.wait()
        if k >= 1:
            p = (k - 1) % 2
            pm = (k - 1) % 3
            for d in g_d[k - 1]:
                d.wait()
            s_d[k - 1] = (
                pltpu.async_copy(v0[p], acc0.at[dstv[pm]], sem_s[p], add=True),
                pltpu.async_copy(v1[p], acc1.at[dstv[pm]], sem_s[p], add=True))
        i_as[k].wait()
        i_bs[k].wait()
        if k + 1 < NCHUNK:
            nb = (k + 1) % 3
            base = w * PER_W + (k + 1) * CHUNK
            i_as[k + 1] = pltpu.async_copy(
                src_hbm.at[pl.ds(base, CHUNK)], srcv[nb], sem_i)
            i_bs[k + 1] = pltpu.async_copy(
                dst_hbm.at[pl.ds(base, CHUNK)], dstv[nb], sem_i)
        g_d[k] = (pltpu.async_copy(tab0.at[srcv[k % 3]], v0[vb], sem_g[vb]),
                  pltpu.async_copy(tab1.at[srcv[k % 3]], v1[vb], sem_g[vb]))
    p = (NCHUNK - 1) % 2
    pm = (NCHUNK - 1) % 3
    for d in g_d[NCHUNK - 1]:
        d.wait()
    s_d[NCHUNK - 1] = (
        pltpu.async_copy(v0[p], acc0.at[dstv[pm]], sem_s[p], add=True),
        pltpu.async_copy(v1[p], acc1.at[dstv[pm]], sem_s[p], add=True))
    for d in s_d[NCHUNK - 2]:
        d.wait()
    for d in s_d[NCHUNK - 1]:
        d.wait()
    plsc.subcore_barrier()
    pltpu.sync_copy(acc0.at[sl], out_hbm.at[c, 0, sl])
    pltpu.sync_copy(acc1.at[sl], out_hbm.at[c, 1, sl])


def _l2_body(src_hbm, dst_hbm, t0_hbm, zeros_hbm, out_hbm,
             tab, acc0, srcv0, srcv1, srcv2, dstv0, dstv1, dstv2, v00, v01,
             sem_i, sem_g0, sem_g1, sem_s0, sem_s1):
    c = lax.axis_index("c")
    s = lax.axis_index("s")
    w = s * NC + c
    sl = pl.ds(s * SLICE, SLICE)
    pltpu.sync_copy(t0_hbm.at[sl], tab.at[sl])
    pltpu.sync_copy(zeros_hbm, acc0.at[sl])
    plsc.subcore_barrier()

    srcv = (srcv0, srcv1, srcv2)
    dstv = (dstv0, dstv1, dstv2)
    v0 = (v00, v01)
    sem_g = (sem_g0, sem_g1)
    sem_s = (sem_s0, sem_s1)

    i_as = [None] * NCHUNK
    i_bs = [None] * NCHUNK
    g_d = [None] * NCHUNK
    s_d = [None] * NCHUNK
    base0 = w * PER_W
    i_as[0] = pltpu.async_copy(src_hbm.at[pl.ds(base0, CHUNK)], srcv[0], sem_i)
    i_bs[0] = pltpu.async_copy(dst_hbm.at[pl.ds(base0, CHUNK)], dstv[0], sem_i)
    for k in range(NCHUNK):
        vb = k % 2
        if k >= 2:
            s_d[k - 2].wait()
        if k >= 1:
            p = (k - 1) % 2
            pm = (k - 1) % 3
            g_d[k - 1].wait()
            s_d[k - 1] = pltpu.async_copy(
                v0[p], acc0.at[dstv[pm]], sem_s[p], add=True)
        i_as[k].wait()
        i_bs[k].wait()
        if k + 1 < NCHUNK:
            nb = (k + 1) % 3
            base = w * PER_W + (k + 1) * CHUNK
            i_as[k + 1] = pltpu.async_copy(
                src_hbm.at[pl.ds(base, CHUNK)], srcv[nb], sem_i)
            i_bs[k + 1] = pltpu.async_copy(
                dst_hbm.at[pl.ds(base, CHUNK)], dstv[nb], sem_i)
        g_d[k] = pltpu.async_copy(tab.at[srcv[k % 3]], v0[vb], sem_g[vb])
    p = (NCHUNK - 1) % 2
    pm = (NCHUNK - 1) % 3
    g_d[NCHUNK - 1].wait()
    s_d[NCHUNK - 1] = pltpu.async_copy(
        v0[p], acc0.at[dstv[pm]], sem_s[p], add=True)
    s_d[NCHUNK - 2].wait()
    s_d[NCHUNK - 1].wait()
    plsc.subcore_barrier()
    pltpu.sync_copy(acc0.at[sl], out_hbm.at[c, sl])


_l1_call = pl.kernel(
    _l1_body,
    out_type=(jax.ShapeDtypeStruct((NC, 2, NPAD), _f32),
              jax.ShapeDtypeStruct((NPAD,), _f32)),
    mesh=_mesh(),
    scratch_types=[
        pltpu.VMEM_SHARED((NPAD,), _f32),
        pltpu.VMEM_SHARED((NPAD,), _f32),
        pltpu.VMEM_SHARED((NPAD,), _f32),
        pltpu.VMEM_SHARED((NPAD,), _f32),
        pltpu.VMEM_SHARED((NPAD,), _f32),
        pltpu.VMEM((CHUNK,), jnp.int32),
        pltpu.VMEM((CHUNK,), jnp.int32),
        pltpu.VMEM((CHUNK,), jnp.int32),
        pltpu.VMEM((CHUNK,), jnp.int32),
        pltpu.VMEM((CHUNK,), jnp.int32),
        pltpu.VMEM((CHUNK,), jnp.int32),
        pltpu.VMEM((CHUNK,), _f32),
        pltpu.VMEM((CHUNK,), _f32),
        pltpu.VMEM((CHUNK,), _f32),
        pltpu.VMEM((CHUNK,), _f32),
        pltpu.VMEM((CHUNK,), _f32),
        pltpu.VMEM((SLICE,), _f32),
        pltpu.VMEM((SLICE,), _f32),
        pltpu.VMEM((SLICE,), _f32),
        pltpu.SemaphoreType.DMA,
        pltpu.SemaphoreType.DMA,
        pltpu.SemaphoreType.DMA,
        pltpu.SemaphoreType.DMA,
        pltpu.SemaphoreType.DMA,
    ],
)

_l2_call = pl.kernel(
    _l2_body,
    out_type=jax.ShapeDtypeStruct((NC, NPAD), _f32),
    mesh=_mesh(),
    scratch_types=[
        pltpu.VMEM_SHARED((NPAD,), _f32),
        pltpu.VMEM_SHARED((NPAD,), _f32),
        pltpu.VMEM((CHUNK,), jnp.int32),
        pltpu.VMEM((CHUNK,), jnp.int32),
        pltpu.VMEM((CHUNK,), jnp.int32),
        pltpu.VMEM((CHUNK,), jnp.int32),
        pltpu.VMEM((CHUNK,), jnp.int32),
        pltpu.VMEM((CHUNK,), jnp.int32),
        pltpu.VMEM((CHUNK,), _f32),
        pltpu.VMEM((CHUNK,), _f32),
        pltpu.SemaphoreType.DMA,
        pltpu.SemaphoreType.DMA,
        pltpu.SemaphoreType.DMA,
        pltpu.SemaphoreType.DMA,
        pltpu.SemaphoreType.DMA,
    ],
)


def _tc2_body(a1p, x0, x1, dinv, w1t, b1, w2t, z_o, y2_o):
    dv = dinv[...]
    d2 = dv * dv
    ap = a1p[0] + a1p[1]
    xx = jnp.concatenate([x0[...], x1[...]], axis=0)
    ax = dv * ap + d2 * xx
    h = jnp.dot(w1t[...], ax, preferred_element_type=_f32) + b1[...]
    h = jnp.maximum(h, 0.0)
    z = jnp.dot(w2t[...], h, preferred_element_type=_f32)
    z_o[...] = z
    y2_o[...] = dv * z


def _tc3_body(a2p, z, dinv, b2, out_o):
    dv = dinv[...]
    out_o[...] = dv * (a2p[0:1, :] + a2p[1:2, :] + dv * z[...]) + b2[...]


_tc2_call = pl.pallas_call(
    _tc2_body,
    out_shape=(
        jax.ShapeDtypeStruct((1, NPAD), _f32),
        jax.ShapeDtypeStruct((1, NPAD), _f32),
    ),
)

_tc3_call = pl.pallas_call(
    _tc3_body,
    out_shape=jax.ShapeDtypeStruct((1, NPAD), _f32),
)


def kernel(x, edge_index, W1, b1, W2, b2):
    src = edge_index[0].astype(jnp.int32)
    dst = edge_index[1].astype(jnp.int32)
    pad = NPAD - N_NODES
    x0 = jnp.pad(x[:, 0], (0, pad)).reshape(1, NPAD)
    x1 = jnp.pad(x[:, 1], (0, pad)).reshape(1, NPAD)
    zeros_h = jnp.zeros((SLICE,), _f32)
    ones_h = jnp.ones((CHUNK,), _f32)
    w1t = W1.T
    w2t = W2.T
    b1c = b1.reshape(64, 1)
    b2c = b2.reshape(1, 1)

    a1p, dinv = _l1_call(src, dst, x0.reshape(NPAD), x1.reshape(NPAD),
                         zeros_h, ones_h)
    dinv2 = dinv.reshape(1, NPAD)
    z, y2 = _tc2_call(a1p, x0, x1, dinv2, w1t, b1c, w2t)
    a2p = _l2_call(src, dst, y2.reshape(NPAD), zeros_h)
    out = _tc3_call(a2p, z, dinv2, b2c)
    return out.reshape(NPAD)[:N_NODES]


# speedup vs baseline: 1.2338x; 1.0578x over previous
import jax
import jax.numpy as jnp
from jax import lax
from jax.experimental import pallas as pl
from jax.experimental.pallas import tpu as pltpu
from jax.experimental.pallas import tpu_sc as plsc

N_NODES = 100000
N_EDGES = 1600000
NPAD = 102400
NC, NS = 2, 16
NW = NC * NS
PER_W = N_EDGES // NW
CHUNK = 5000
NCHUNK = PER_W // CHUNK
SLICE = NPAD // NS

_f32 = jnp.float32


def _mesh():
    return plsc.VectorSubcoreMesh(
        core_axis_name="c", subcore_axis_name="s", num_cores=NC, num_subcores=NS
    )


DEG_PER_TILE = N_EDGES // NS
DEG_NCHUNK = DEG_PER_TILE // CHUNK


def _nrsqrt16(x):
    i = lax.bitcast_convert_type(x, jnp.int32)
    i = 0x5F3759DF - (i >> 1)
    y = lax.bitcast_convert_type(i, _f32)
    for _ in range(3):
        y = y * (1.5 - 0.5 * x * y * y)
    return y


def _l1_body(src_hbm, dst_hbm, x0_hbm, x1_hbm, zeros_hbm, ones_hbm,
             out_hbm, dinv_hbm,
             dacc, tab0, tab1, acc0, acc1,
             srcv0, srcv1, srcv2, dstv0, dstv1, dstv2,
             v00, v01, v10, v11, onesv, vdeg, vx0, vx1,
             sem_i, sem_g0, sem_g1, sem_s0, sem_s1):
    c = lax.axis_index("c")
    s = lax.axis_index("s")
    w = s * NC + c
    sl = pl.ds(s * SLICE, SLICE)
    dbuf = (dstv0, dstv1, dstv2)
    sem_s = (sem_s0, sem_s1)

    pltpu.sync_copy(zeros_hbm, dacc.at[sl])
    pltpu.sync_copy(ones_hbm, onesv)
    plsc.subcore_barrier()
    i_d = [None] * DEG_NCHUNK
    d_d = [None] * DEG_NCHUNK
    i_d[0] = pltpu.async_copy(
        dst_hbm.at[pl.ds(s * DEG_PER_TILE, CHUNK)], dbuf[0], sem_i)
    for k in range(DEG_NCHUNK):
        i_d[k].wait()
        if k >= 2:
            d_d[k - 2].wait()
        if k + 1 < DEG_NCHUNK:
            base = s * DEG_PER_TILE + (k + 1) * CHUNK
            i_d[k + 1] = pltpu.async_copy(
                dst_hbm.at[pl.ds(base, CHUNK)], dbuf[(k + 1) % 3], sem_i)
        d_d[k] = pltpu.async_copy(onesv, dacc.at[dbuf[k % 3]], sem_s[k % 2],
                                  add=True)
    d_d[DEG_NCHUNK - 2].wait()
    d_d[DEG_NCHUNK - 1].wait()
    plsc.subcore_barrier()

    pltpu.sync_copy(dacc.at[sl], vdeg)
    pltpu.sync_copy(x0_hbm.at[sl], vx0)
    pltpu.sync_copy(x1_hbm.at[sl], vx1)

    def pbody(i, carry):
        ds16 = pl.ds(i * 16, 16)
        dv = _nrsqrt16(vdeg[ds16] + 1.0)
        vdeg[ds16] = dv
        vx0[ds16] = dv * vx0[ds16]
        vx1[ds16] = dv * vx1[ds16]
        return carry

    lax.fori_loop(0, SLICE // 16, pbody, 0)
    pltpu.sync_copy(vx0, tab0.at[sl])
    pltpu.sync_copy(vx1, tab1.at[sl])

    @pl.when(c == 0)
    def _():
        pltpu.sync_copy(vdeg, dinv_hbm.at[sl])

    pltpu.sync_copy(zeros_hbm, acc0.at[sl])
    pltpu.sync_copy(zeros_hbm, acc1.at[sl])
    plsc.subcore_barrier()

    srcv = (srcv0, srcv1, srcv2)
    dstv = (dstv0, dstv1, dstv2)
    v0 = (v00, v01)
    v1 = (v10, v11)
    sem_g = (sem_g0, sem_g1)
    sem_s = (sem_s0, sem_s1)

    i_as = [None] * NCHUNK
    i_bs = [None] * NCHUNK
    g_d = [None] * NCHUNK
    s_d = [None] * NCHUNK
    base0 = w * PER_W
    i_as[0] = pltpu.async_copy(src_hbm.at[pl.ds(base0, CHUNK)], srcv[0], sem_i)
    i_bs[0] = pltpu.async_copy(dst_hbm.at[pl.ds(base0, CHUNK)], dstv[0], sem_i)
    for k in range(NCHUNK):
        vb = k % 2
        if k >= 2:
            for d in s_d[k - 2]:
                d.wait()
        if k >= 1:
            p = (k - 1) % 2
            pm = (k - 1) % 3
            for d in g_d[k - 1]:
                d.wait()
            s_d[k - 1] = (
                pltpu.async_copy(v0[p], acc0.at[dstv[pm]], sem_s[p], add=True),
                pltpu.async_copy(v1[p], acc1.at[dstv[pm]], sem_s[p], add=True))
        i_as[k].wait()
        i_bs[k].wait()
        if k + 1 < NCHUNK:
            nb = (k + 1) % 3
            base = w * PER_W + (k + 1) * CHUNK
            i_as[k + 1] = pltpu.async_copy(
                src_hbm.at[pl.ds(base, CHUNK)], srcv[nb], sem_i)
            i_bs[k + 1] = pltpu.async_copy(
                dst_hbm.at[pl.ds(base, CHUNK)], dstv[nb], sem_i)
        g_d[k] = (pltpu.async_copy(tab0.at[srcv[k % 3]], v0[vb], sem_g[vb]),
                  pltpu.async_copy(tab1.at[srcv[k % 3]], v1[vb], sem_g[vb]))
    p = (NCHUNK - 1) % 2
    pm = (NCHUNK - 1) % 3
    for d in g_d[NCHUNK - 1]:
        d.wait()
    s_d[NCHUNK - 1] = (
        pltpu.async_copy(v0[p], acc0.at[dstv[pm]], sem_s[p], add=True),
        pltpu.async_copy(v1[p], acc1.at[dstv[pm]], sem_s[p], add=True))
    for d in s_d[NCHUNK - 2]:
        d.wait()
    for d in s_d[NCHUNK - 1]:
        d.wait()
    plsc.subcore_barrier()
    pltpu.sync_copy(acc0.at[sl], out_hbm.at[c, 0, sl])
    pltpu.sync_copy(acc1.at[sl], out_hbm.at[c, 1, sl])


def _l2_body(src_hbm, dst_hbm, t0_hbm, zeros_hbm, out_hbm,
             tab, acc0, srcv0, srcv1, srcv2, dstv0, dstv1, dstv2, v00, v01,
             sem_i, sem_g0, sem_g1, sem_s0, sem_s1):
    c = lax.axis_index("c")
    s = lax.axis_index("s")
    w = s * NC + c
    sl = pl.ds(s * SLICE, SLICE)
    pltpu.sync_copy(t0_hbm.at[sl], tab.at[sl])
    pltpu.sync_copy(zeros_hbm, acc0.at[sl])
    plsc.subcore_barrier()

    srcv = (srcv0, srcv1, srcv2)
    dstv = (dstv0, dstv1, dstv2)
    v0 = (v00, v01)
    sem_g = (sem_g0, sem_g1)
    sem_s = (sem_s0, sem_s1)

    i_as = [None] * NCHUNK
    i_bs = [None] * NCHUNK
    g_d = [None] * NCHUNK
    s_d = [None] * NCHUNK
    base0 = w * PER_W
    i_as[0] = pltpu.async_copy(src_hbm.at[pl.ds(base0, CHUNK)], srcv[0], sem_i)
    i_bs[0] = pltpu.async_copy(dst_hbm.at[pl.ds(base0, CHUNK)], dstv[0], sem_i)
    for k in range(NCHUNK):
        vb = k % 2
        if k >= 2:
            s_d[k - 2].wait()
        if k >= 1:
            p = (k - 1) % 2
            pm = (k - 1) % 3
            g_d[k - 1].wait()
            s_d[k - 1] = pltpu.async_copy(
                v0[p], acc0.at[dstv[pm]], sem_s[p], add=True)
        i_as[k].wait()
        i_bs[k].wait()
        if k + 1 < NCHUNK:
            nb = (k + 1) % 3
            base = w * PER_W + (k + 1) * CHUNK
            i_as[k + 1] = pltpu.async_copy(
                src_hbm.at[pl.ds(base, CHUNK)], srcv[nb], sem_i)
            i_bs[k + 1] = pltpu.async_copy(
                dst_hbm.at[pl.ds(base, CHUNK)], dstv[nb], sem_i)
        g_d[k] = pltpu.async_copy(tab.at[srcv[k % 3]], v0[vb], sem_g[vb])
    p = (NCHUNK - 1) % 2
    pm = (NCHUNK - 1) % 3
    g_d[NCHUNK - 1].wait()
    s_d[NCHUNK - 1] = pltpu.async_copy(
        v0[p], acc0.at[dstv[pm]], sem_s[p], add=True)
    s_d[NCHUNK - 2].wait()
    s_d[NCHUNK - 1].wait()
    plsc.subcore_barrier()
    pltpu.sync_copy(acc0.at[sl], out_hbm.at[c, sl])


_l1_call = pl.kernel(
    _l1_body,
    out_type=(jax.ShapeDtypeStruct((NC, 2, NPAD), _f32),
              jax.ShapeDtypeStruct((NPAD,), _f32)),
    mesh=_mesh(),
    scratch_types=[
        pltpu.VMEM_SHARED((NPAD,), _f32),
        pltpu.VMEM_SHARED((NPAD,), _f32),
        pltpu.VMEM_SHARED((NPAD,), _f32),
        pltpu.VMEM_SHARED((NPAD,), _f32),
        pltpu.VMEM_SHARED((NPAD,), _f32),
        pltpu.VMEM((CHUNK,), jnp.int32),
        pltpu.VMEM((CHUNK,), jnp.int32),
        pltpu.VMEM((CHUNK,), jnp.int32),
        pltpu.VMEM((CHUNK,), jnp.int32),
        pltpu.VMEM((CHUNK,), jnp.int32),
        pltpu.VMEM((CHUNK,), jnp.int32),
        pltpu.VMEM((CHUNK,), _f32),
        pltpu.VMEM((CHUNK,), _f32),
        pltpu.VMEM((CHUNK,), _f32),
        pltpu.VMEM((CHUNK,), _f32),
        pltpu.VMEM((CHUNK,), _f32),
        pltpu.VMEM((SLICE,), _f32),
        pltpu.VMEM((SLICE,), _f32),
        pltpu.VMEM((SLICE,), _f32),
        pltpu.SemaphoreType.DMA,
        pltpu.SemaphoreType.DMA,
        pltpu.SemaphoreType.DMA,
        pltpu.SemaphoreType.DMA,
        pltpu.SemaphoreType.DMA,
    ],
)

_l2_call = pl.kernel(
    _l2_body,
    out_type=jax.ShapeDtypeStruct((NC, NPAD), _f32),
    mesh=_mesh(),
    scratch_types=[
        pltpu.VMEM_SHARED((NPAD,), _f32),
        pltpu.VMEM_SHARED((NPAD,), _f32),
        pltpu.VMEM((CHUNK,), jnp.int32),
        pltpu.VMEM((CHUNK,), jnp.int32),
        pltpu.VMEM((CHUNK,), jnp.int32),
        pltpu.VMEM((CHUNK,), jnp.int32),
        pltpu.VMEM((CHUNK,), jnp.int32),
        pltpu.VMEM((CHUNK,), jnp.int32),
        pltpu.VMEM((CHUNK,), _f32),
        pltpu.VMEM((CHUNK,), _f32),
        pltpu.SemaphoreType.DMA,
        pltpu.SemaphoreType.DMA,
        pltpu.SemaphoreType.DMA,
        pltpu.SemaphoreType.DMA,
        pltpu.SemaphoreType.DMA,
    ],
)


def _tc2_body(a1p, x0, x1, dinv, w1t, b1, w2t, z_o, y2_o):
    dv = dinv[...]
    d2 = dv * dv
    ap = a1p[0] + a1p[1]
    xx = jnp.concatenate([x0[...], x1[...]], axis=0)
    ax = dv * ap + d2 * xx
    h = jnp.dot(w1t[...], ax, preferred_element_type=_f32) + b1[...]
    h = jnp.maximum(h, 0.0)
    z = jnp.dot(w2t[...], h, preferred_element_type=_f32)
    z_o[...] = z
    y2_o[...] = dv * z


def _tc3_body(a2p, z, dinv, b2, out_o):
    dv = dinv[...]
    out_o[...] = dv * (a2p[0:1, :] + a2p[1:2, :] + dv * z[...]) + b2[...]


_tc2_call = pl.pallas_call(
    _tc2_body,
    out_shape=(
        jax.ShapeDtypeStruct((1, NPAD), _f32),
        jax.ShapeDtypeStruct((1, NPAD), _f32),
    ),
)

_tc3_call = pl.pallas_call(
    _tc3_body,
    out_shape=jax.ShapeDtypeStruct((1, NPAD), _f32),
)


def kernel(x, edge_index, W1, b1, W2, b2):
    src = edge_index[0].astype(jnp.int32)
    dst = edge_index[1].astype(jnp.int32)
    pad = NPAD - N_NODES
    x0 = jnp.pad(x[:, 0], (0, pad)).reshape(1, NPAD)
    x1 = jnp.pad(x[:, 1], (0, pad)).reshape(1, NPAD)
    zeros_h = jnp.zeros((SLICE,), _f32)
    ones_h = jnp.ones((CHUNK,), _f32)
    w1t = W1.T
    w2t = W2.T
    b1c = b1.reshape(64, 1)
    b2c = b2.reshape(1, 1)

    a1p, dinv = _l1_call(src, dst, x0.reshape(NPAD), x1.reshape(NPAD),
                         zeros_h, ones_h)
    dinv2 = dinv.reshape(1, NPAD)
    z, y2 = _tc2_call(a1p, x0, x1, dinv2, w1t, b1c, w2t)
    a2p = _l2_call(src, dst, y2.reshape(NPAD), zeros_h)
    out = _tc3_call(a2p, z, dinv2, b2c)
    return out.reshape(NPAD)[:N_NODES]


# speedup vs baseline: 1.2778x; 1.0356x over previous
import jax
import jax.numpy as jnp
from jax import lax
from jax.experimental import pallas as pl
from jax.experimental.pallas import tpu as pltpu
from jax.experimental.pallas import tpu_sc as plsc

N_NODES = 100000
N_EDGES = 1600000
NPAD = 102400
NC, NS = 2, 16
NW = NC * NS
PER_W = N_EDGES // NW
CHUNK = 5000
NCHUNK = PER_W // CHUNK
SLICE = NPAD // NS

_f32 = jnp.float32


def _mesh():
    return plsc.VectorSubcoreMesh(
        core_axis_name="c", subcore_axis_name="s", num_cores=NC, num_subcores=NS
    )


DEG_PER_TILE = N_EDGES // NS
DEG_NCHUNK = DEG_PER_TILE // CHUNK


def _nrsqrt16(x):
    i = lax.bitcast_convert_type(x, jnp.int32)
    i = 0x5F3759DF - (i >> 1)
    y = lax.bitcast_convert_type(i, _f32)
    for _ in range(2):
        y = y * (1.5 - 0.5 * x * y * y)
    return y


def _l1_body(src_hbm, dst_hbm, x0_hbm, x1_hbm, zeros_hbm, ones_hbm,
             out_hbm, dinv_hbm,
             dacc, tab0, tab1, acc0, acc1,
             srcv0, srcv1, srcv2, dstv0, dstv1, dstv2,
             v00, v01, v10, v11, onesv, vdeg, vx0, vx1,
             sem_i, sem_g0, sem_g1, sem_s0, sem_s1):
    c = lax.axis_index("c")
    s = lax.axis_index("s")
    w = s * NC + c
    sl = pl.ds(s * SLICE, SLICE)
    dbuf = (dstv0, dstv1, dstv2)
    sem_s = (sem_s0, sem_s1)

    pltpu.sync_copy(zeros_hbm, dacc.at[sl])
    pltpu.sync_copy(ones_hbm, onesv)
    pre = (pltpu.async_copy(x0_hbm.at[sl], vx0, sem_g0),
           pltpu.async_copy(x1_hbm.at[sl], vx1, sem_g0),
           pltpu.async_copy(zeros_hbm, acc0.at[sl], sem_g0),
           pltpu.async_copy(zeros_hbm, acc1.at[sl], sem_g0))
    plsc.subcore_barrier()
    i_d = [None] * DEG_NCHUNK
    d_d = [None] * DEG_NCHUNK
    i_d[0] = pltpu.async_copy(
        dst_hbm.at[pl.ds(s * DEG_PER_TILE, CHUNK)], dbuf[0], sem_i)
    for k in range(DEG_NCHUNK):
        i_d[k].wait()
        if k >= 2:
            d_d[k - 2].wait()
        if k + 1 < DEG_NCHUNK:
            base = s * DEG_PER_TILE + (k + 1) * CHUNK
            i_d[k + 1] = pltpu.async_copy(
                dst_hbm.at[pl.ds(base, CHUNK)], dbuf[(k + 1) % 3], sem_i)
        d_d[k] = pltpu.async_copy(onesv, dacc.at[dbuf[k % 3]], sem_s[k % 2],
                                  add=True)
    d_d[DEG_NCHUNK - 2].wait()
    d_d[DEG_NCHUNK - 1].wait()
    srcv = (srcv0, srcv1, srcv2)
    dstv = (dstv0, dstv1, dstv2)
    i_as = [None] * NCHUNK
    i_bs = [None] * NCHUNK
    base0 = w * PER_W
    i_as[0] = pltpu.async_copy(src_hbm.at[pl.ds(base0, CHUNK)], srcv[0], sem_i)
    i_bs[0] = pltpu.async_copy(dst_hbm.at[pl.ds(base0, CHUNK)], dstv[0], sem_i)
    plsc.subcore_barrier()

    for d in pre:
        d.wait()
    pltpu.sync_copy(dacc.at[sl], vdeg)

    def pbody(i, carry):
        ds16 = pl.ds(i * 16, 16)
        dv = _nrsqrt16(vdeg[ds16] + 1.0)
        vdeg[ds16] = dv
        vx0[ds16] = dv * vx0[ds16]
        vx1[ds16] = dv * vx1[ds16]
        return carry

    lax.fori_loop(0, SLICE // 16, pbody, 0)
    t_d = (pltpu.async_copy(vx0, tab0.at[sl], sem_g0),
           pltpu.async_copy(vx1, tab1.at[sl], sem_g1))

    @pl.when(c == 0)
    def _():
        pltpu.sync_copy(vdeg, dinv_hbm.at[sl])

    for d in t_d:
        d.wait()
    plsc.subcore_barrier()

    v0 = (v00, v01)
    v1 = (v10, v11)
    sem_g = (sem_g0, sem_g1)
    sem_s = (sem_s0, sem_s1)

    g_d = [None] * NCHUNK
    s_d = [None] * NCHUNK
    for k in range(NCHUNK):
        vb = k % 2
        if k >= 2:
            for d in s_d[k - 2]:
                d.wait()
        if k >= 1:
            p = (k - 1) % 2
            pm = (k - 1) % 3
            for d in g_d[k - 1]:
                d.wait()
            s_d[k - 1] = (
                pltpu.async_copy(v0[p], acc0.at[dstv[pm]], sem_s[p], add=True),
                pltpu.async_copy(v1[p], acc1.at[dstv[pm]], sem_s[p], add=True))
        i_as[k].wait()
        i_bs[k].wait()
        if k + 1 < NCHUNK:
            nb = (k + 1) % 3
            base = w * PER_W + (k + 1) * CHUNK
            i_as[k + 1] = pltpu.async_copy(
                src_hbm.at[pl.ds(base, CHUNK)], srcv[nb], sem_i)
            i_bs[k + 1] = pltpu.async_copy(
                dst_hbm.at[pl.ds(base, CHUNK)], dstv[nb], sem_i)
        g_d[k] = (pltpu.async_copy(tab0.at[srcv[k % 3]], v0[vb], sem_g[vb]),
                  pltpu.async_copy(tab1.at[srcv[k % 3]], v1[vb], sem_g[vb]))
    p = (NCHUNK - 1) % 2
    pm = (NCHUNK - 1) % 3
    for d in g_d[NCHUNK - 1]:
        d.wait()
    s_d[NCHUNK - 1] = (
        pltpu.async_copy(v0[p], acc0.at[dstv[pm]], sem_s[p], add=True),
        pltpu.async_copy(v1[p], acc1.at[dstv[pm]], sem_s[p], add=True))
    for d in s_d[NCHUNK - 2]:
        d.wait()
    for d in s_d[NCHUNK - 1]:
        d.wait()
    plsc.subcore_barrier()
    pltpu.sync_copy(acc0.at[sl], out_hbm.at[c, 0, sl])
    pltpu.sync_copy(acc1.at[sl], out_hbm.at[c, 1, sl])


def _l2_body(src_hbm, dst_hbm, t0_hbm, zeros_hbm, out_hbm,
             tab, acc0, srcv0, srcv1, srcv2, dstv0, dstv1, dstv2, v00, v01,
             sem_i, sem_g0, sem_g1, sem_s0, sem_s1):
    c = lax.axis_index("c")
    s = lax.axis_index("s")
    w = s * NC + c
    sl = pl.ds(s * SLICE, SLICE)
    srcv = (srcv0, srcv1, srcv2)
    dstv = (dstv0, dstv1, dstv2)
    i_as = [None] * NCHUNK
    i_bs = [None] * NCHUNK
    base0 = w * PER_W
    i_as[0] = pltpu.async_copy(src_hbm.at[pl.ds(base0, CHUNK)], srcv[0], sem_i)
    i_bs[0] = pltpu.async_copy(dst_hbm.at[pl.ds(base0, CHUNK)], dstv[0], sem_i)
    t_d = (pltpu.async_copy(t0_hbm.at[sl], tab.at[sl], sem_g0),
           pltpu.async_copy(zeros_hbm, acc0.at[sl], sem_g1))
    for d in t_d:
        d.wait()
    plsc.subcore_barrier()

    v0 = (v00, v01)
    sem_g = (sem_g0, sem_g1)
    sem_s = (sem_s0, sem_s1)

    g_d = [None] * NCHUNK
    s_d = [None] * NCHUNK
    for k in range(NCHUNK):
        vb = k % 2
        if k >= 2:
            s_d[k - 2].wait()
        if k >= 1:
            p = (k - 1) % 2
            pm = (k - 1) % 3
            g_d[k - 1].wait()
            s_d[k - 1] = pltpu.async_copy(
                v0[p], acc0.at[dstv[pm]], sem_s[p], add=True)
        i_as[k].wait()
        i_bs[k].wait()
        if k + 1 < NCHUNK:
            nb = (k + 1) % 3
            base = w * PER_W + (k + 1) * CHUNK
            i_as[k + 1] = pltpu.async_copy(
                src_hbm.at[pl.ds(base, CHUNK)], srcv[nb], sem_i)
            i_bs[k + 1] = pltpu.async_copy(
                dst_hbm.at[pl.ds(base, CHUNK)], dstv[nb], sem_i)
        g_d[k] = pltpu.async_copy(tab.at[srcv[k % 3]], v0[vb], sem_g[vb])
    p = (NCHUNK - 1) % 2
    pm = (NCHUNK - 1) % 3
    g_d[NCHUNK - 1].wait()
    s_d[NCHUNK - 1] = pltpu.async_copy(
        v0[p], acc0.at[dstv[pm]], sem_s[p], add=True)
    s_d[NCHUNK - 2].wait()
    s_d[NCHUNK - 1].wait()
    plsc.subcore_barrier()
    pltpu.sync_copy(acc0.at[sl], out_hbm.at[c, sl])


_l1_call = pl.kernel(
    _l1_body,
    out_type=(jax.ShapeDtypeStruct((NC, 2, NPAD), _f32),
              jax.ShapeDtypeStruct((NPAD,), _f32)),
    mesh=_mesh(),
    scratch_types=[
        pltpu.VMEM_SHARED((NPAD,), _f32),
        pltpu.VMEM_SHARED((NPAD,), _f32),
        pltpu.VMEM_SHARED((NPAD,), _f32),
        pltpu.VMEM_SHARED((NPAD,), _f32),
        pltpu.VMEM_SHARED((NPAD,), _f32),
        pltpu.VMEM((CHUNK,), jnp.int32),
        pltpu.VMEM((CHUNK,), jnp.int32),
        pltpu.VMEM((CHUNK,), jnp.int32),
        pltpu.VMEM((CHUNK,), jnp.int32),
        pltpu.VMEM((CHUNK,), jnp.int32),
        pltpu.VMEM((CHUNK,), jnp.int32),
        pltpu.VMEM((CHUNK,), _f32),
        pltpu.VMEM((CHUNK,), _f32),
        pltpu.VMEM((CHUNK,), _f32),
        pltpu.VMEM((CHUNK,), _f32),
        pltpu.VMEM((CHUNK,), _f32),
        pltpu.VMEM((SLICE,), _f32),
        pltpu.VMEM((SLICE,), _f32),
        pltpu.VMEM((SLICE,), _f32),
        pltpu.SemaphoreType.DMA,
        pltpu.SemaphoreType.DMA,
        pltpu.SemaphoreType.DMA,
        pltpu.SemaphoreType.DMA,
        pltpu.SemaphoreType.DMA,
    ],
)

_l2_call = pl.kernel(
    _l2_body,
    out_type=jax.ShapeDtypeStruct((NC, NPAD), _f32),
    mesh=_mesh(),
    scratch_types=[
        pltpu.VMEM_SHARED((NPAD,), _f32),
        pltpu.VMEM_SHARED((NPAD,), _f32),
        pltpu.VMEM((CHUNK,), jnp.int32),
        pltpu.VMEM((CHUNK,), jnp.int32),
        pltpu.VMEM((CHUNK,), jnp.int32),
        pltpu.VMEM((CHUNK,), jnp.int32),
        pltpu.VMEM((CHUNK,), jnp.int32),
        pltpu.VMEM((CHUNK,), jnp.int32),
        pltpu.VMEM((CHUNK,), _f32),
        pltpu.VMEM((CHUNK,), _f32),
        pltpu.SemaphoreType.DMA,
        pltpu.SemaphoreType.DMA,
        pltpu.SemaphoreType.DMA,
        pltpu.SemaphoreType.DMA,
        pltpu.SemaphoreType.DMA,
    ],
)


def _tc2_body(a1p, x0, x1, dinv, w1t, b1, w2t, z_o, y2_o):
    dv = dinv[...]
    d2 = dv * dv
    ap = a1p[0] + a1p[1]
    xx = jnp.concatenate([x0[...], x1[...]], axis=0)
    ax = dv * ap + d2 * xx
    h = jnp.dot(w1t[...], ax, preferred_element_type=_f32) + b1[...]
    h = jnp.maximum(h, 0.0)
    z = jnp.dot(w2t[...], h, preferred_element_type=_f32)
    z_o[...] = z
    y2_o[...] = dv * z


def _tc3_body(a2p, z, dinv, b2, out_o):
    dv = dinv[...]
    out_o[...] = dv * (a2p[0:1, :] + a2p[1:2, :] + dv * z[...]) + b2[...]


_tc2_call = pl.pallas_call(
    _tc2_body,
    out_shape=(
        jax.ShapeDtypeStruct((1, NPAD), _f32),
        jax.ShapeDtypeStruct((1, NPAD), _f32),
    ),
)

_tc3_call = pl.pallas_call(
    _tc3_body,
    out_shape=jax.ShapeDtypeStruct((1, NPAD), _f32),
)


def kernel(x, edge_index, W1, b1, W2, b2):
    src = edge_index[0].astype(jnp.int32)
    dst = edge_index[1].astype(jnp.int32)
    pad = NPAD - N_NODES
    x0 = jnp.pad(x[:, 0], (0, pad)).reshape(1, NPAD)
    x1 = jnp.pad(x[:, 1], (0, pad)).reshape(1, NPAD)
    zeros_h = jnp.zeros((SLICE,), _f32)
    ones_h = jnp.ones((CHUNK,), _f32)
    w1t = W1.T
    w2t = W2.T
    b1c = b1.reshape(64, 1)
    b2c = b2.reshape(1, 1)

    a1p, dinv = _l1_call(src, dst, x0.reshape(NPAD), x1.reshape(NPAD),
                         zeros_h, ones_h)
    dinv2 = dinv.reshape(1, NPAD)
    z, y2 = _tc2_call(a1p, x0, x1, dinv2, w1t, b1c, w2t)
    a2p = _l2_call(src, dst, y2.reshape(NPAD), zeros_h)
    out = _tc3_call(a2p, z, dinv2, b2c)
    return out.reshape(NPAD)[:N_NODES]


# speedup vs baseline: 1.2855x; 1.0060x over previous
import jax
import jax.numpy as jnp
from jax import lax
from jax.experimental import pallas as pl
from jax.experimental.pallas import tpu as pltpu
from jax.experimental.pallas import tpu_sc as plsc

N_NODES = 100000
N_EDGES = 1600000
NPAD = 102400
NC, NS = 2, 16
NW = NC * NS
PER_W = N_EDGES // NW
CHUNK = 5000
NCHUNK = PER_W // CHUNK
SLICE = NPAD // NS
CHUNK2 = 10000
NCHUNK2 = PER_W // CHUNK2

_f32 = jnp.float32


def _mesh():
    return plsc.VectorSubcoreMesh(
        core_axis_name="c", subcore_axis_name="s", num_cores=NC, num_subcores=NS
    )


DEG_PER_TILE = N_EDGES // NS
DEG_NCHUNK = DEG_PER_TILE // CHUNK


def _nrsqrt16(x):
    i = lax.bitcast_convert_type(x, jnp.int32)
    i = 0x5F3759DF - (i >> 1)
    y = lax.bitcast_convert_type(i, _f32)
    for _ in range(2):
        y = y * (1.5 - 0.5 * x * y * y)
    return y


def _l1_body(src_hbm, dst_hbm, x0_hbm, x1_hbm, zeros_hbm, ones_hbm,
             out_hbm, dinv_hbm,
             dacc, tab0, tab1, acc0, acc1,
             srcv0, srcv1, srcv2, dstv0, dstv1, dstv2,
             v00, v01, v10, v11, onesv, vdeg, vx0, vx1,
             sem_i, sem_g0, sem_g1, sem_s0, sem_s1):
    c = lax.axis_index("c")
    s = lax.axis_index("s")
    w = s * NC + c
    sl = pl.ds(s * SLICE, SLICE)
    dbuf = (dstv0, dstv1, dstv2)
    sem_s = (sem_s0, sem_s1)

    pltpu.sync_copy(zeros_hbm, dacc.at[sl])
    pltpu.sync_copy(ones_hbm, onesv)
    pre = (pltpu.async_copy(x0_hbm.at[sl], vx0, sem_g0),
           pltpu.async_copy(x1_hbm.at[sl], vx1, sem_g0),
           pltpu.async_copy(zeros_hbm, acc0.at[sl], sem_g0),
           pltpu.async_copy(zeros_hbm, acc1.at[sl], sem_g0))
    plsc.subcore_barrier()
    i_d = [None] * DEG_NCHUNK
    d_d = [None] * DEG_NCHUNK
    i_d[0] = pltpu.async_copy(
        dst_hbm.at[pl.ds(s * DEG_PER_TILE, CHUNK)], dbuf[0], sem_i)
    for k in range(DEG_NCHUNK):
        i_d[k].wait()
        if k >= 2:
            d_d[k - 2].wait()
        if k + 1 < DEG_NCHUNK:
            base = s * DEG_PER_TILE + (k + 1) * CHUNK
            i_d[k + 1] = pltpu.async_copy(
                dst_hbm.at[pl.ds(base, CHUNK)], dbuf[(k + 1) % 3], sem_i)
        d_d[k] = pltpu.async_copy(onesv, dacc.at[dbuf[k % 3]], sem_s[k % 2],
                                  add=True)
    d_d[DEG_NCHUNK - 2].wait()
    d_d[DEG_NCHUNK - 1].wait()
    srcv = (srcv0, srcv1, srcv2)
    dstv = (dstv0, dstv1, dstv2)
    i_as = [None] * NCHUNK
    i_bs = [None] * NCHUNK
    base0 = w * PER_W
    i_as[0] = pltpu.async_copy(src_hbm.at[pl.ds(base0, CHUNK)], srcv[0], sem_i)
    i_bs[0] = pltpu.async_copy(dst_hbm.at[pl.ds(base0, CHUNK)], dstv[0], sem_i)
    plsc.subcore_barrier()

    for d in pre:
        d.wait()
    pltpu.sync_copy(dacc.at[sl], vdeg)

    def pbody(i, carry):
        ds16 = pl.ds(i * 16, 16)
        dv = _nrsqrt16(vdeg[ds16] + 1.0)
        vdeg[ds16] = dv
        vx0[ds16] = dv * vx0[ds16]
        vx1[ds16] = dv * vx1[ds16]
        return carry

    lax.fori_loop(0, SLICE // 16, pbody, 0)
    t_d = (pltpu.async_copy(vx0, tab0.at[sl], sem_g0),
           pltpu.async_copy(vx1, tab1.at[sl], sem_g1))

    @pl.when(c == 0)
    def _():
        pltpu.sync_copy(vdeg, dinv_hbm.at[sl])

    for d in t_d:
        d.wait()
    plsc.subcore_barrier()

    v0 = (v00, v01)
    v1 = (v10, v11)
    sem_g = (sem_g0, sem_g1)
    sem_s = (sem_s0, sem_s1)

    g_d = [None] * NCHUNK
    s_d = [None] * NCHUNK
    for k in range(NCHUNK):
        vb = k % 2
        if k >= 2:
            for d in s_d[k - 2]:
                d.wait()
        if k >= 1:
            p = (k - 1) % 2
            pm = (k - 1) % 3
            for d in g_d[k - 1]:
                d.wait()
            s_d[k - 1] = (
                pltpu.async_copy(v0[p], acc0.at[dstv[pm]], sem_s[p], add=True),
                pltpu.async_copy(v1[p], acc1.at[dstv[pm]], sem_s[p], add=True))
        i_as[k].wait()
        i_bs[k].wait()
        if k + 1 < NCHUNK:
            nb = (k + 1) % 3
            base = w * PER_W + (k + 1) * CHUNK
            i_as[k + 1] = pltpu.async_copy(
                src_hbm.at[pl.ds(base, CHUNK)], srcv[nb], sem_i)
            i_bs[k + 1] = pltpu.async_copy(
                dst_hbm.at[pl.ds(base, CHUNK)], dstv[nb], sem_i)
        g_d[k] = (pltpu.async_copy(tab0.at[srcv[k % 3]], v0[vb], sem_g[vb]),
                  pltpu.async_copy(tab1.at[srcv[k % 3]], v1[vb], sem_g[vb]))
    p = (NCHUNK - 1) % 2
    pm = (NCHUNK - 1) % 3
    for d in g_d[NCHUNK - 1]:
        d.wait()
    s_d[NCHUNK - 1] = (
        pltpu.async_copy(v0[p], acc0.at[dstv[pm]], sem_s[p], add=True),
        pltpu.async_copy(v1[p], acc1.at[dstv[pm]], sem_s[p], add=True))
    for d in s_d[NCHUNK - 2]:
        d.wait()
    for d in s_d[NCHUNK - 1]:
        d.wait()
    plsc.subcore_barrier()
    pltpu.sync_copy(acc0.at[sl], out_hbm.at[c, 0, sl])
    pltpu.sync_copy(acc1.at[sl], out_hbm.at[c, 1, sl])


def _l2_body(src_hbm, dst_hbm, t0_hbm, zeros_hbm, out_hbm,
             tab, acc0, srcv0, srcv1, srcv2, dstv0, dstv1, dstv2, v00, v01,
             sem_i, sem_g0, sem_g1, sem_s0, sem_s1):
    c = lax.axis_index("c")
    s = lax.axis_index("s")
    w = s * NC + c
    sl = pl.ds(s * SLICE, SLICE)
    srcv = (srcv0, srcv1, srcv2)
    dstv = (dstv0, dstv1, dstv2)
    i_as = [None] * NCHUNK2
    i_bs = [None] * NCHUNK2
    base0 = w * PER_W
    i_as[0] = pltpu.async_copy(src_hbm.at[pl.ds(base0, CHUNK2)], srcv[0], sem_i)
    i_bs[0] = pltpu.async_copy(dst_hbm.at[pl.ds(base0, CHUNK2)], dstv[0], sem_i)
    t_d = (pltpu.async_copy(t0_hbm.at[sl], tab.at[sl], sem_g0),
           pltpu.async_copy(zeros_hbm, acc0.at[sl], sem_g1))
    for d in t_d:
        d.wait()
    plsc.subcore_barrier()

    v0 = (v00, v01)
    sem_g = (sem_g0, sem_g1)
    sem_s = (sem_s0, sem_s1)

    g_d = [None] * NCHUNK2
    s_d = [None] * NCHUNK2
    for k in range(NCHUNK2):
        vb = k % 2
        if k >= 2:
            s_d[k - 2].wait()
        if k >= 1:
            p = (k - 1) % 2
            pm = (k - 1) % 3
            g_d[k - 1].wait()
            s_d[k - 1] = pltpu.async_copy(
                v0[p], acc0.at[dstv[pm]], sem_s[p], add=True)
        i_as[k].wait()
        i_bs[k].wait()
        if k + 1 < NCHUNK2:
            nb = (k + 1) % 3
            base = w * PER_W + (k + 1) * CHUNK2
            i_as[k + 1] = pltpu.async_copy(
                src_hbm.at[pl.ds(base, CHUNK2)], srcv[nb], sem_i)
            i_bs[k + 1] = pltpu.async_copy(
                dst_hbm.at[pl.ds(base, CHUNK2)], dstv[nb], sem_i)
        g_d[k] = pltpu.async_copy(tab.at[srcv[k % 3]], v0[vb], sem_g[vb])
    p = (NCHUNK2 - 1) % 2
    pm = (NCHUNK2 - 1) % 3
    g_d[NCHUNK2 - 1].wait()
    s_d[NCHUNK2 - 1] = pltpu.async_copy(
        v0[p], acc0.at[dstv[pm]], sem_s[p], add=True)
    s_d[NCHUNK2 - 2].wait()
    s_d[NCHUNK2 - 1].wait()
    plsc.subcore_barrier()
    pltpu.sync_copy(acc0.at[sl], out_hbm.at[c, sl])


_l1_call = pl.kernel(
    _l1_body,
    out_type=(jax.ShapeDtypeStruct((NC, 2, NPAD), _f32),
              jax.ShapeDtypeStruct((NPAD,), _f32)),
    mesh=_mesh(),
    scratch_types=[
        pltpu.VMEM_SHARED((NPAD,), _f32),
        pltpu.VMEM_SHARED((NPAD,), _f32),
        pltpu.VMEM_SHARED((NPAD,), _f32),
        pltpu.VMEM_SHARED((NPAD,), _f32),
        pltpu.VMEM_SHARED((NPAD,), _f32),
        pltpu.VMEM((CHUNK,), jnp.int32),
        pltpu.VMEM((CHUNK,), jnp.int32),
        pltpu.VMEM((CHUNK,), jnp.int32),
        pltpu.VMEM((CHUNK,), jnp.int32),
        pltpu.VMEM((CHUNK,), jnp.int32),
        pltpu.VMEM((CHUNK,), jnp.int32),
        pltpu.VMEM((CHUNK,), _f32),
        pltpu.VMEM((CHUNK,), _f32),
        pltpu.VMEM((CHUNK,), _f32),
        pltpu.VMEM((CHUNK,), _f32),
        pltpu.VMEM((CHUNK,), _f32),
        pltpu.VMEM((SLICE,), _f32),
        pltpu.VMEM((SLICE,), _f32),
        pltpu.VMEM((SLICE,), _f32),
        pltpu.SemaphoreType.DMA,
        pltpu.SemaphoreType.DMA,
        pltpu.SemaphoreType.DMA,
        pltpu.SemaphoreType.DMA,
        pltpu.SemaphoreType.DMA,
    ],
)

_l2_call = pl.kernel(
    _l2_body,
    out_type=jax.ShapeDtypeStruct((NC, NPAD), _f32),
    mesh=_mesh(),
    scratch_types=[
        pltpu.VMEM_SHARED((NPAD,), _f32),
        pltpu.VMEM_SHARED((NPAD,), _f32),
        pltpu.VMEM((CHUNK2,), jnp.int32),
        pltpu.VMEM((CHUNK2,), jnp.int32),
        pltpu.VMEM((CHUNK2,), jnp.int32),
        pltpu.VMEM((CHUNK2,), jnp.int32),
        pltpu.VMEM((CHUNK2,), jnp.int32),
        pltpu.VMEM((CHUNK2,), jnp.int32),
        pltpu.VMEM((CHUNK2,), _f32),
        pltpu.VMEM((CHUNK2,), _f32),
        pltpu.SemaphoreType.DMA,
        pltpu.SemaphoreType.DMA,
        pltpu.SemaphoreType.DMA,
        pltpu.SemaphoreType.DMA,
        pltpu.SemaphoreType.DMA,
    ],
)


def _tc2_body(a1p, x0, x1, dinv, w1t, b1, w2t, z_o, y2_o):
    dv = dinv[...]
    d2 = dv * dv
    ap = a1p[0] + a1p[1]
    xx = jnp.concatenate([x0[...], x1[...]], axis=0)
    ax = dv * ap + d2 * xx
    h = jnp.dot(w1t[...], ax, preferred_element_type=_f32) + b1[...]
    h = jnp.maximum(h, 0.0)
    z = jnp.dot(w2t[...], h, preferred_element_type=_f32)
    z_o[...] = z
    y2_o[...] = dv * z


def _tc3_body(a2p, z, dinv, b2, out_o):
    dv = dinv[...]
    out_o[...] = dv * (a2p[0:1, :] + a2p[1:2, :] + dv * z[...]) + b2[...]


_tc2_call = pl.pallas_call(
    _tc2_body,
    out_shape=(
        jax.ShapeDtypeStruct((1, NPAD), _f32),
        jax.ShapeDtypeStruct((1, NPAD), _f32),
    ),
)

_tc3_call = pl.pallas_call(
    _tc3_body,
    out_shape=jax.ShapeDtypeStruct((1, NPAD), _f32),
)


def kernel(x, edge_index, W1, b1, W2, b2):
    src = edge_index[0].astype(jnp.int32)
    dst = edge_index[1].astype(jnp.int32)
    pad = NPAD - N_NODES
    x0 = jnp.pad(x[:, 0], (0, pad)).reshape(1, NPAD)
    x1 = jnp.pad(x[:, 1], (0, pad)).reshape(1, NPAD)
    zeros_h = jnp.zeros((SLICE,), _f32)
    ones_h = jnp.ones((CHUNK,), _f32)
    w1t = W1.T
    w2t = W2.T
    b1c = b1.reshape(64, 1)
    b2c = b2.reshape(1, 1)

    a1p, dinv = _l1_call(src, dst, x0.reshape(NPAD), x1.reshape(NPAD),
                         zeros_h, ones_h)
    dinv2 = dinv.reshape(1, NPAD)
    z, y2 = _tc2_call(a1p, x0, x1, dinv2, w1t, b1c, w2t)
    a2p = _l2_call(src, dst, y2.reshape(NPAD), zeros_h)
    out = _tc3_call(a2p, z, dinv2, b2c)
    return out.reshape(NPAD)[:N_NODES]
